# Initial kernel scaffold; baseline (speedup 1.0000x reference)
import jax
import jax.numpy as jnp
from jax.experimental import pallas as pl


def kernel(node_states, edges, vertices, me_w1, me_b1, me_w2, me_b2, me_w3, me_b3, nu_w1, nu_b1, nu_w2, nu_b2, nu_w3, nu_b3):
    raise NotImplementedError("write your pallas kernel here")


# speedup vs baseline: 5.9631x; 5.9631x over previous
import functools

import jax
import jax.numpy as jnp
from jax import lax
from jax.experimental import pallas as pl
from jax.experimental.pallas import tpu as pltpu
from jax.experimental.pallas import tpu_sc as plsc

F32 = jnp.float32

SC_CORES = 2
SC_SUBCORES = 16
SC_WORKERS = SC_CORES * SC_SUBCORES
CHUNK = 128


def _dot(a, b):
    return jnp.dot(a, b, preferred_element_type=F32)


def _ab_tables(ns, w1a, w1b, bn=2000):
    n, d = ns.shape
    mh = w1a.shape[1]

    def body(ns_ref, wa_ref, wb_ref, a_ref, b_ref):
        a_ref[...] = _dot(ns_ref[...], wa_ref[...])
        b_ref[...] = _dot(ns_ref[...], wb_ref[...])

    return pl.pallas_call(
        body,
        grid=(n // bn,),
        in_specs=[
            pl.BlockSpec((bn, d), lambda i: (i, 0)),
            pl.BlockSpec((d, mh), lambda i: (0, 0)),
            pl.BlockSpec((d, mh), lambda i: (0, 0)),
        ],
        out_specs=[
            pl.BlockSpec((bn, mh), lambda i: (i, 0)),
            pl.BlockSpec((bn, mh), lambda i: (i, 0)),
        ],
        out_shape=[
            jax.ShapeDtypeStruct((n, mh), F32),
            jax.ShapeDtypeStruct((n, mh), F32),
        ],
    )(ns, w1a, w1b)


def _sc_gather(a_tab, b_tab, vi, vj):
    n, d = a_tab.shape
    e = vi.shape[0]
    nch = e // CHUNK
    tmax = -(-nch // SC_WORKERS)
    mesh = plsc.VectorSubcoreMesh(core_axis_name="c", subcore_axis_name="s")

    @functools.partial(
        pl.kernel,
        mesh=mesh,
        out_type=(
            jax.ShapeDtypeStruct((e, d), F32),
            jax.ShapeDtypeStruct((e, d), F32),
        ),
        scratch_types=[
            pltpu.VMEM((CHUNK,), jnp.int32),
            pltpu.VMEM((CHUNK,), jnp.int32),
            pltpu.VMEM((CHUNK, d), F32),
            pltpu.VMEM((CHUNK, d), F32),
            pltpu.SemaphoreType.DMA,
            pltpu.SemaphoreType.DMA,
        ],
    )
    def gather_k(a_hbm, b_hbm, vi_hbm, vj_hbm, ga_hbm, gb_hbm, ii, jj, ba, bb, s0, s1):
        c = lax.axis_index("c")
        s = lax.axis_index("s")
        w = s * SC_CORES + c

        def body(t, carry):
            cid = t * SC_WORKERS + w

            @pl.when(cid < nch)
            def _():
                base = cid * CHUNK
                pltpu.sync_copy(vi_hbm.at[pl.ds(base, CHUNK)], ii)
                pltpu.sync_copy(vj_hbm.at[pl.ds(base, CHUNK)], jj)
                da = pltpu.async_copy(a_hbm.at[ii], ba, s0)
                db = pltpu.async_copy(b_hbm.at[jj], bb, s1)
                da.wait()
                db.wait()
                pltpu.sync_copy(ba, ga_hbm.at[pl.ds(base, CHUNK)])
                pltpu.sync_copy(bb, gb_hbm.at[pl.ds(base, CHUNK)])

            return carry

        lax.fori_loop(0, tmax, body, 0)

    return gather_k(a_tab, b_tab, vi, vj)


def _edge_mlp(ga, gb, edges, w1e, b1, w2, b2, w3, b3, be=2560):
    e, d = ga.shape
    de = edges.shape[1]
    mh = w2.shape[0]
    md = w3.shape[1]

    def body(ga_ref, gb_ref, e_ref, w1e_ref, b1_ref, w2_ref, b2_ref, w3_ref, b3_ref, out_ref):
        h = ga_ref[...] + gb_ref[...] + _dot(e_ref[...], w1e_ref[...]) + b1_ref[...]
        h = jnp.maximum(h, 0.0)
        h = jnp.maximum(_dot(h, w2_ref[...]) + b2_ref[...], 0.0)
        out_ref[...] = _dot(h, w3_ref[...]) + b3_ref[...]

    return pl.pallas_call(
        body,
        grid=(e // be,),
        in_specs=[
            pl.BlockSpec((be, d), lambda i: (i, 0)),
            pl.BlockSpec((be, d), lambda i: (i, 0)),
            pl.BlockSpec((be, de), lambda i: (i, 0)),
            pl.BlockSpec((de, mh), lambda i: (0, 0)),
            pl.BlockSpec((1, mh), lambda i: (0, 0)),
            pl.BlockSpec((mh, mh), lambda i: (0, 0)),
            pl.BlockSpec((1, mh), lambda i: (0, 0)),
            pl.BlockSpec((mh, md), lambda i: (0, 0)),
            pl.BlockSpec((1, md), lambda i: (0, 0)),
        ],
        out_specs=pl.BlockSpec((be, md), lambda i: (i, 0)),
        out_shape=jax.ShapeDtypeStruct((e, md), F32),
    )(ga, gb, edges, w1e, b1, w2, b2, w3, b3)


def _sc_scatter(msg, vi, vj, zeros, n):
    e, md = msg.shape
    nch = e // CHUNK
    tmax = -(-nch // SC_WORKERS)
    rows = -(-n // SC_SUBCORES)
    rows = (rows + 7) // 8 * 8
    last_rows = n - (SC_SUBCORES - 1) * rows
    mesh = plsc.VectorSubcoreMesh(core_axis_name="c", subcore_axis_name="s")

    @functools.partial(
        pl.kernel,
        mesh=mesh,
        out_type=jax.ShapeDtypeStruct((SC_CORES * n, md), F32),
        scratch_types=[
            pltpu.VMEM((CHUNK,), jnp.int32),
            pltpu.VMEM((CHUNK,), jnp.int32),
            pltpu.VMEM((CHUNK, md), F32),
            pltpu.VMEM_SHARED((n, md), F32),
        ],
    )
    def scatter_k(msg_hbm, vi_hbm, vj_hbm, zero_hbm, out_hbm, ii, jj, mv, acc):
        c = lax.axis_index("c")
        s = lax.axis_index("s")
        w = s * SC_CORES + c

        @pl.when(s < SC_SUBCORES - 1)
        def _():
            pltpu.sync_copy(zero_hbm.at[pl.ds(s * rows, rows)],
                            acc.at[pl.ds(s * rows, rows)])

        @pl.when(s == SC_SUBCORES - 1)
        def _():
            pltpu.sync_copy(zero_hbm.at[pl.ds(s * rows, last_rows)],
                            acc.at[pl.ds(s * rows, last_rows)])

        plsc.subcore_barrier()

        def body(t, carry):
            cid = t * SC_WORKERS + w

            @pl.when(cid < nch)
            def _():
                base = cid * CHUNK
                pltpu.sync_copy(vi_hbm.at[pl.ds(base, CHUNK)], ii)
                pltpu.sync_copy(vj_hbm.at[pl.ds(base, CHUNK)], jj)
                pltpu.sync_copy(msg_hbm.at[pl.ds(base, CHUNK)], mv)
                pltpu.sync_copy(mv, acc.at[ii], add=True)
                pltpu.sync_copy(mv, acc.at[jj], add=True)

            return carry

        lax.fori_loop(0, tmax, body, 0)
        plsc.subcore_barrier()

        @pl.when(s < SC_SUBCORES - 1)
        def _():
            pltpu.sync_copy(acc.at[pl.ds(s * rows, rows)],
                            out_hbm.at[pl.ds(c * n + s * rows, rows)])

        @pl.when(s == SC_SUBCORES - 1)
        def _():
            pltpu.sync_copy(acc.at[pl.ds(s * rows, last_rows)],
                            out_hbm.at[pl.ds(c * n + s * rows, last_rows)])

    return scatter_k(msg, vi, vj, zeros)


def _node_mlp(ns, ns_perm, s0, s1, w1a, w1b, w1c, b1, w2, b2, w3, b3, bn=2000):
    n, d = ns.shape
    uh = w2.shape[0]

    def body(ns_ref, np_ref, s0_ref, s1_ref, w1a_ref, w1b_ref, w1c_ref, b1_ref,
             w2_ref, b2_ref, w3_ref, b3_ref, out_ref):
        att = ns_ref[...] - np_ref[...]
        summed = s0_ref[...] + s1_ref[...]
        u = (_dot(ns_ref[...], w1a_ref[...]) + _dot(summed, w1b_ref[...])
             + _dot(att, w1c_ref[...]) + b1_ref[...])
        u = jnp.maximum(u, 0.0)
        u = jnp.maximum(_dot(u, w2_ref[...]) + b2_ref[...], 0.0)
        out_ref[...] = _dot(u, w3_ref[...]) + b3_ref[...]

    row = lambda i: (i, 0)
    full = lambda i: (0, 0)
    return pl.pallas_call(
        body,
        grid=(n // bn,),
        in_specs=[
            pl.BlockSpec((bn, d), row),
            pl.BlockSpec((bn, d), row),
            pl.BlockSpec((bn, d), row),
            pl.BlockSpec((bn, d), row),
            pl.BlockSpec((d, uh), full),
            pl.BlockSpec((d, uh), full),
            pl.BlockSpec((d, uh), full),
            pl.BlockSpec((1, uh), full),
            pl.BlockSpec((uh, uh), full),
            pl.BlockSpec((1, uh), full),
            pl.BlockSpec((uh, d), full),
            pl.BlockSpec((1, d), full),
        ],
        out_specs=pl.BlockSpec((bn, d), row),
        out_shape=jax.ShapeDtypeStruct((n, d), F32),
    )(ns, ns_perm, s0, s1, w1a, w1b, w1c, b1, w2, b2, w3, b3)


def kernel(node_states, edges, vertices, me_w1, me_b1, me_w2, me_b2, me_w3,
           me_b3, nu_w1, nu_b1, nu_w2, nu_b2, nu_w3, nu_b3):
    n, d = node_states.shape
    e, de = edges.shape
    md = me_w3.shape[1]

    vi = vertices[:, 0]
    vj = vertices[:, 1]

    w1a = me_w1[:d]
    w1b = me_w1[d:2 * d]
    w1e = me_w1[2 * d:]

    a_tab, b_tab = _ab_tables(node_states, w1a, w1b)
    ga, gb = _sc_gather(a_tab, b_tab, vi, vj)
    msg = _edge_mlp(ga, gb, edges, w1e, me_b1.reshape(1, -1), me_w2,
                    me_b2.reshape(1, -1), me_w3, me_b3.reshape(1, -1))
    zeros = jnp.zeros((n, md), F32)
    partials = _sc_scatter(msg, vi, vj, zeros, n)
    s0 = partials[:n]
    s1 = partials[n:]

    half = n // 4
    ns_perm = node_states.reshape(2, 2, half, d)[:, ::-1].reshape(n, d)

    uw1a = nu_w1[:d]
    uw1b = nu_w1[d:d + md]
    uw1c = nu_w1[d + md:]
    return _node_mlp(node_states, ns_perm, s0, s1, uw1a, uw1b, uw1c,
                     nu_b1.reshape(1, -1), nu_w2, nu_b2.reshape(1, -1),
                     nu_w3, nu_b3.reshape(1, -1))


# speedup vs baseline: 6.5365x; 1.0962x over previous
import functools

import jax
import jax.numpy as jnp
from jax import lax
from jax.experimental import pallas as pl
from jax.experimental.pallas import tpu as pltpu
from jax.experimental.pallas import tpu_sc as plsc

F32 = jnp.float32

SC_CORES = 2
SC_SUBCORES = 16
SC_WORKERS = SC_CORES * SC_SUBCORES
CHUNK = 128


def _dot(a, b):
    return jnp.dot(a, b, preferred_element_type=F32)


def _ab_tables(ns, w1a, w1b, bn=2000):
    n, d = ns.shape
    mh = w1a.shape[1]

    def body(ns_ref, wa_ref, wb_ref, a_ref, b_ref):
        a_ref[...] = _dot(ns_ref[...], wa_ref[...])
        b_ref[...] = _dot(ns_ref[...], wb_ref[...])

    return pl.pallas_call(
        body,
        grid=(n // bn,),
        in_specs=[
            pl.BlockSpec((bn, d), lambda i: (i, 0)),
            pl.BlockSpec((d, mh), lambda i: (0, 0)),
            pl.BlockSpec((d, mh), lambda i: (0, 0)),
        ],
        out_specs=[
            pl.BlockSpec((bn, mh), lambda i: (i, 0)),
            pl.BlockSpec((bn, mh), lambda i: (i, 0)),
        ],
        out_shape=[
            jax.ShapeDtypeStruct((n, mh), F32),
            jax.ShapeDtypeStruct((n, mh), F32),
        ],
    )(ns, w1a, w1b)


def _sc_gather(a_tab, b_tab, vi, vj):
    n, d = a_tab.shape
    dt = a_tab.dtype
    e = vi.shape[0]
    nch = e // CHUNK
    tmax = -(-nch // SC_WORKERS)
    mesh = plsc.VectorSubcoreMesh(core_axis_name="c", subcore_axis_name="s")

    @functools.partial(
        pl.kernel,
        mesh=mesh,
        out_type=(
            jax.ShapeDtypeStruct((e, d), dt),
            jax.ShapeDtypeStruct((e, d), dt),
        ),
        scratch_types=[
            pltpu.VMEM((CHUNK,), jnp.int32),
            pltpu.VMEM((CHUNK,), jnp.int32),
            pltpu.VMEM((CHUNK, d), dt),
            pltpu.VMEM((CHUNK, d), dt),
            pltpu.SemaphoreType.DMA,
            pltpu.SemaphoreType.DMA,
        ],
    )
    def gather_k(a_hbm, b_hbm, vi_hbm, vj_hbm, ga_hbm, gb_hbm, ii, jj, ba, bb, s0, s1):
        c = lax.axis_index("c")
        s = lax.axis_index("s")
        w = s * SC_CORES + c

        def body(t, carry):
            cid = t * SC_WORKERS + w

            @pl.when(cid < nch)
            def _():
                base = cid * CHUNK
                pltpu.sync_copy(vi_hbm.at[pl.ds(base, CHUNK)], ii)
                pltpu.sync_copy(vj_hbm.at[pl.ds(base, CHUNK)], jj)
                da = pltpu.async_copy(a_hbm.at[ii], ba, s0)
                db = pltpu.async_copy(b_hbm.at[jj], bb, s1)
                da.wait()
                db.wait()
                pltpu.sync_copy(ba, ga_hbm.at[pl.ds(base, CHUNK)])
                pltpu.sync_copy(bb, gb_hbm.at[pl.ds(base, CHUNK)])

            return carry

        lax.fori_loop(0, tmax, body, 0)

    return gather_k(a_tab, b_tab, vi, vj)


def _edge_mlp(ga, gb, edges, w1e, b1, w2, b2, w3, b3, be=2000):
    e, d = ga.shape
    de = edges.shape[1]
    mh = w2.shape[0]
    md = w3.shape[1]

    def body(ga_ref, gb_ref, e_ref, w1e_ref, b1_ref, w2_ref, b2_ref, w3_ref, b3_ref, out_ref):
        h = (ga_ref[...] + gb_ref[...]
             + _dot(e_ref[...], w1e_ref[...]) + b1_ref[...])
        h = jnp.maximum(h, 0.0).astype(jnp.bfloat16)
        h = jnp.maximum(_dot(h, w2_ref[...]) + b2_ref[...], 0.0).astype(jnp.bfloat16)
        out_ref[...] = _dot(h, w3_ref[...]) + b3_ref[...]

    return pl.pallas_call(
        body,
        grid=(e // be,),
        in_specs=[
            pl.BlockSpec((be, d), lambda i: (i, 0)),
            pl.BlockSpec((be, d), lambda i: (i, 0)),
            pl.BlockSpec((be, de), lambda i: (i, 0)),
            pl.BlockSpec((de, mh), lambda i: (0, 0)),
            pl.BlockSpec((1, mh), lambda i: (0, 0)),
            pl.BlockSpec((mh, mh), lambda i: (0, 0)),
            pl.BlockSpec((1, mh), lambda i: (0, 0)),
            pl.BlockSpec((mh, md), lambda i: (0, 0)),
            pl.BlockSpec((1, md), lambda i: (0, 0)),
        ],
        out_specs=pl.BlockSpec((be, md), lambda i: (i, 0)),
        out_shape=jax.ShapeDtypeStruct((e, md), F32),
    )(ga, gb, edges, w1e, b1, w2, b2, w3, b3)


def _sc_scatter(msgs, vis, vjs, zeros, n):
    nparts = len(msgs)
    e, md = msgs[0].shape
    nch = e // CHUNK
    tmax = -(-nch // SC_WORKERS)
    rows = -(-n // SC_SUBCORES)
    rows = (rows + 7) // 8 * 8
    last_rows = n - (SC_SUBCORES - 1) * rows
    mesh = plsc.VectorSubcoreMesh(core_axis_name="c", subcore_axis_name="s")

    @functools.partial(
        pl.kernel,
        mesh=mesh,
        out_type=jax.ShapeDtypeStruct((SC_CORES * n, md), F32),
        scratch_types=[
            pltpu.VMEM((CHUNK,), jnp.int32),
            pltpu.VMEM((CHUNK,), jnp.int32),
            pltpu.VMEM((CHUNK, md), F32),
            pltpu.VMEM_SHARED((n, md), F32),
        ],
    )
    def scatter_k(*refs):
        msg_hbms = refs[:nparts]
        vi_hbms = refs[nparts:2 * nparts]
        vj_hbms = refs[2 * nparts:3 * nparts]
        zero_hbm = refs[3 * nparts]
        out_hbm = refs[3 * nparts + 1]
        ii, jj, mv, acc = refs[3 * nparts + 2:]
        c = lax.axis_index("c")
        s = lax.axis_index("s")
        w = s * SC_CORES + c

        @pl.when(s < SC_SUBCORES - 1)
        def _():
            pltpu.sync_copy(zero_hbm.at[pl.ds(s * rows, rows)],
                            acc.at[pl.ds(s * rows, rows)])

        @pl.when(s == SC_SUBCORES - 1)
        def _():
            pltpu.sync_copy(zero_hbm.at[pl.ds(s * rows, last_rows)],
                            acc.at[pl.ds(s * rows, last_rows)])

        plsc.subcore_barrier()

        for p in range(nparts):
            msg_hbm, vi_hbm, vj_hbm = msg_hbms[p], vi_hbms[p], vj_hbms[p]

            def body(t, carry):
                cid = t * SC_WORKERS + w

                @pl.when(cid < nch)
                def _():
                    base = cid * CHUNK
                    pltpu.sync_copy(vi_hbm.at[pl.ds(base, CHUNK)], ii)
                    pltpu.sync_copy(vj_hbm.at[pl.ds(base, CHUNK)], jj)
                    pltpu.sync_copy(msg_hbm.at[pl.ds(base, CHUNK)], mv)
                    pltpu.sync_copy(mv, acc.at[ii], add=True)
                    pltpu.sync_copy(mv, acc.at[jj], add=True)

                return carry

            lax.fori_loop(0, tmax, body, 0)

        plsc.subcore_barrier()

        @pl.when(s < SC_SUBCORES - 1)
        def _():
            pltpu.sync_copy(acc.at[pl.ds(s * rows, rows)],
                            out_hbm.at[pl.ds(c * n + s * rows, rows)])

        @pl.when(s == SC_SUBCORES - 1)
        def _():
            pltpu.sync_copy(acc.at[pl.ds(s * rows, last_rows)],
                            out_hbm.at[pl.ds(c * n + s * rows, last_rows)])

    return scatter_k(*msgs, *vis, *vjs, zeros)


def _node_mlp(ns, ns_perm, parts, w1a, w1b, w1c, b1, w2, b2, w3, b3, bn=2000):
    n, d = ns.shape
    uh = w2.shape[0]
    np_ = len(parts)

    def body(*refs):
        ns_ref, np_ref = refs[0], refs[1]
        part_refs = refs[2:2 + np_]
        (w1a_ref, w1b_ref, w1c_ref, b1_ref, w2_ref, b2_ref, w3_ref,
         b3_ref, out_ref) = refs[2 + np_:]
        att = ns_ref[...] - np_ref[...]
        summed = part_refs[0][...]
        for pr in part_refs[1:]:
            summed = summed + pr[...]
        u = (_dot(ns_ref[...], w1a_ref[...]) + _dot(summed, w1b_ref[...])
             + _dot(att, w1c_ref[...]) + b1_ref[...])
        u = jnp.maximum(u, 0.0)
        u = jnp.maximum(_dot(u, w2_ref[...]) + b2_ref[...], 0.0)
        out_ref[...] = _dot(u, w3_ref[...]) + b3_ref[...]

    row = lambda i: (i, 0)
    full = lambda i: (0, 0)
    return pl.pallas_call(
        body,
        grid=(n // bn,),
        in_specs=[
            pl.BlockSpec((bn, d), row),
            pl.BlockSpec((bn, d), row),
        ] + [pl.BlockSpec((bn, d), row) for _ in range(np_)] + [
            pl.BlockSpec((d, uh), full),
            pl.BlockSpec((d, uh), full),
            pl.BlockSpec((d, uh), full),
            pl.BlockSpec((1, uh), full),
            pl.BlockSpec((uh, uh), full),
            pl.BlockSpec((1, uh), full),
            pl.BlockSpec((uh, d), full),
            pl.BlockSpec((1, d), full),
        ],
        out_specs=pl.BlockSpec((bn, d), row),
        out_shape=jax.ShapeDtypeStruct((n, d), F32),
    )(ns, ns_perm, *parts, w1a, w1b, w1c, b1, w2, b2, w3, b3)


def kernel(node_states, edges, vertices, me_w1, me_b1, me_w2, me_b2, me_w3,
           me_b3, nu_w1, nu_b1, nu_w2, nu_b2, nu_w3, nu_b3):
    n, d = node_states.shape
    e, de = edges.shape
    md = me_w3.shape[1]

    vi = vertices[:, 0]
    vj = vertices[:, 1]

    w1a = me_w1[:d]
    w1b = me_w1[d:2 * d]
    w1e = me_w1[2 * d:]

    a_tab, b_tab = _ab_tables(node_states, w1a, w1b)

    nsplit = 4
    es = e // nsplit
    bf = jnp.bfloat16
    w2b, w3b = me_w2.astype(bf), me_w3.astype(bf)
    b1r, b2r, b3r = (me_b1.reshape(1, -1), me_b2.reshape(1, -1),
                     me_b3.reshape(1, -1))
    msgs, vis, vjs = [], [], []
    for k in range(nsplit):
        sl = slice(k * es, (k + 1) * es)
        vik, vjk = vi[sl], vj[sl]
        gak, gbk = _sc_gather(a_tab, b_tab, vik, vjk)
        msgs.append(_edge_mlp(gak, gbk, edges[sl], w1e, b1r, w2b, b2r,
                              w3b, b3r))
        vis.append(vik)
        vjs.append(vjk)

    zeros = jnp.zeros((n, md), F32)
    pa = _sc_scatter(msgs[:2], vis[:2], vjs[:2], zeros, n)
    pb = _sc_scatter(msgs[2:], vis[2:], vjs[2:], zeros, n)
    parts = [pa[:n], pa[n:], pb[:n], pb[n:]]

    half = n // 4
    ns_perm = node_states.reshape(2, 2, half, d)[:, ::-1].reshape(n, d)

    uw1a = nu_w1[:d]
    uw1b = nu_w1[d:d + md]
    uw1c = nu_w1[d + md:]
    return _node_mlp(node_states, ns_perm, parts, uw1a, uw1b, uw1c,
                     nu_b1.reshape(1, -1), nu_w2, nu_b2.reshape(1, -1),
                     nu_w3, nu_b3.reshape(1, -1))
